# Initial kernel scaffold; baseline (speedup 1.0000x reference)
import functools

import jax
import jax.numpy as jnp
from jax import lax
from jax.experimental import pallas as pl
from jax.experimental.pallas import tpu as pltpu
from jax.experimental.pallas import tpu_sc as plsc

V = 1000000
D = 64
BLOCK = 4096
N_BLOCKS = (V * D + BLOCK - 1) // BLOCK
AM_PAD = 16384

NC = 2
NS = 16
NW = NC * NS
L = 16

B_TOTAL = 16384 * 50
PER_TILE = B_TOTAL // NW
CHUNK = 128
NCHUNK = PER_TILE // CHUNK
GROUPS = CHUNK // L


def _rsqrt_newton(x):
    half = x * jnp.full((L,), 0.5, jnp.float32)
    i = plsc.bitcast(x, jnp.int32)
    i = jnp.full((L,), 0x5F3759DF, jnp.int32) - lax.shift_right_logical(
        i, jnp.full((L,), 1, jnp.int32))
    y = plsc.bitcast(i, jnp.float32)
    three_half = jnp.full((L,), 1.5, jnp.float32)
    for _ in range(3):
        y = y * (three_half - half * y * y)
    return y


def _body(x_hbm, w_hbm, am_hbm, code_hbm, lnw_hbm, lnb_hbm, out_hbm,
          idx_c, am_v, code_v, lnw_v, lnb_v, rows_v, out_v, mstat, rstat,
          gsem):
    c = lax.axis_index("c")
    s = lax.axis_index("s")
    wid = s * NC + c

    pltpu.sync_copy(am_hbm, am_v)
    pltpu.sync_copy(code_hbm, code_v)
    pltpu.sync_copy(lnw_hbm, lnw_v)
    pltpu.sync_copy(lnb_hbm, lnb_v)

    iota = lax.iota(jnp.int32, L)
    inv_d = jnp.full((L,), 1.0 / D, jnp.float32)
    eps = jnp.full((L,), 1e-5, jnp.float32)
    lnw_regs = tuple(lnw_v[pl.ds(sp * L, L)] for sp in range(D // L))
    lnb_regs = tuple(lnb_v[pl.ds(sp * L, L)] for sp in range(D // L))

    @pl.loop(0, NCHUNK)
    def chunk(j):
        pltpu.sync_copy(x_hbm.at[wid, j], idx_c)
        pltpu.async_copy(w_hbm.at[idx_c], rows_v, gsem).wait()

        @pl.loop(0, GROUPS)
        def group(g):
            rvec = g * L + iota
            idxv = idx_c[pl.ds(g * L, L)]
            am = plsc.load_gather(
                am_v, [lax.shift_right_logical(idxv, jnp.full((L,), 6, jnp.int32))])
            acc = [jnp.zeros((L,), jnp.float32) for _ in range(4)]
            acc2 = [jnp.zeros((L,), jnp.float32) for _ in range(4)]
            for d in range(D):
                dfull = jnp.full((L,), d, jnp.int32)
                cvec = plsc.load_gather(rows_v, [rvec, dfull])
                v = plsc.load_gather(code_v, [cvec]) * am
                plsc.store_scatter(out_v, [rvec, dfull], v)
                a = d & 3
                acc[a] = acc[a] + v
                acc2[a] = acc2[a] + v * v
            tot = (acc[0] + acc[1]) + (acc[2] + acc[3])
            tot2 = (acc2[0] + acc2[1]) + (acc2[2] + acc2[3])
            mean = tot * inv_d
            var = tot2 * inv_d - mean * mean
            rstd = _rsqrt_newton(var + eps)
            mstat[pl.ds(g * L, L)] = mean
            rstat[pl.ds(g * L, L)] = rstd

        @pl.loop(0, CHUNK)
        def rowfix(r):
            rfull = jnp.full((L,), r, jnp.int32)
            mv = plsc.load_gather(mstat, [rfull])
            rv = plsc.load_gather(rstat, [rfull])
            for sp in range(D // L):
                dv = out_v[r, pl.ds(sp * L, L)]
                out_v[r, pl.ds(sp * L, L)] = (
                    (dv - mv) * rv * lnw_regs[sp] + lnb_regs[sp])

        pltpu.sync_copy(out_v, out_hbm.at[pl.ds(wid * PER_TILE + j * CHUNK, CHUNK)])


@jax.jit
def _run(x3, weight, am_pad, code, ln_weight, ln_bias):
    mesh = plsc.VectorSubcoreMesh(core_axis_name="c", subcore_axis_name="s")
    return pl.kernel(
        _body,
        out_type=jax.ShapeDtypeStruct((B_TOTAL, D), jnp.float32),
        mesh=mesh,
        scratch_types=[
            pltpu.VMEM((CHUNK,), jnp.int32),
            pltpu.VMEM((AM_PAD,), jnp.float32),
            pltpu.VMEM((256,), jnp.float32),
            pltpu.VMEM((D,), jnp.float32),
            pltpu.VMEM((D,), jnp.float32),
            pltpu.VMEM((CHUNK, D), jnp.int32),
            pltpu.VMEM((CHUNK, D), jnp.float32),
            pltpu.VMEM((CHUNK,), jnp.float32),
            pltpu.VMEM((CHUNK,), jnp.float32),
            pltpu.SemaphoreType.DMA,
        ],
    )(x3, weight, am_pad, code, ln_weight, ln_bias)


def kernel(x, weight, absmax, code, ln_weight, ln_bias):
    xs = x.shape
    x3 = x.reshape(NW, NCHUNK, CHUNK)
    am_pad = jnp.concatenate(
        [absmax, jnp.zeros((AM_PAD - N_BLOCKS,), jnp.float32)])
    out = _run(x3, weight, am_pad, code, ln_weight, ln_bias)
    return out.reshape(xs[0], xs[1], D)


# speedup vs baseline: 296.6958x; 296.6958x over previous
import functools

import jax
import jax.numpy as jnp
from jax import lax
from jax.experimental import pallas as pl
from jax.experimental.pallas import tpu as pltpu
from jax.experimental.pallas import tpu_sc as plsc

V = 1000000
D = 64
BLOCK = 4096
N_BLOCKS = (V * D + BLOCK - 1) // BLOCK
AM_PAD = 16384

NC = 2
NS = 16
NW = NC * NS
L = 16

B_TOTAL = 16384 * 50
PER_TILE = B_TOTAL // NW
CHUNK = 128
NCHUNK = PER_TILE // CHUNK
GROUPS = CHUNK // L


def _rsqrt_newton(x):
    half = x * jnp.full((L,), 0.5, jnp.float32)
    i = plsc.bitcast(x, jnp.int32)
    i = jnp.full((L,), 0x5F3759DF, jnp.int32) - lax.shift_right_logical(
        i, jnp.full((L,), 1, jnp.int32))
    y = plsc.bitcast(i, jnp.float32)
    three_half = jnp.full((L,), 1.5, jnp.float32)
    for _ in range(3):
        y = y * (three_half - half * y * y)
    return y


def _body(x_hbm, w_hbm, am_hbm, code_hbm, lnw_hbm, lnb_hbm, out_hbm,
          idx_c, am_v, code_v, lnw_v, lnb_v, rows_v, out_v, mstat, rstat,
          gsem):
    c = lax.axis_index("c")
    s = lax.axis_index("s")
    wid = s * NC + c

    pltpu.sync_copy(am_hbm, am_v)
    pltpu.sync_copy(code_hbm, code_v)
    pltpu.sync_copy(lnw_hbm, lnw_v)
    pltpu.sync_copy(lnb_hbm, lnb_v)

    iota = lax.iota(jnp.int32, L)
    inv_d = jnp.full((L,), 1.0 / D, jnp.float32)
    eps = jnp.full((L,), 1e-5, jnp.float32)
    lnw_regs = tuple(lnw_v[pl.ds(sp * L, L)] for sp in range(D // L))
    lnb_regs = tuple(lnb_v[pl.ds(sp * L, L)] for sp in range(D // L))

    @pl.loop(0, NCHUNK)
    def chunk(j):
        pltpu.sync_copy(x_hbm.at[wid, j], idx_c)
        pltpu.async_copy(w_hbm.at[idx_c], rows_v, gsem).wait()

        @pl.loop(0, GROUPS)
        def group(g):
            rvec = g * L + iota
            idxv = idx_c[pl.ds(g * L, L)]
            am = plsc.load_gather(
                am_v, [lax.shift_right_logical(idxv, jnp.full((L,), 6, jnp.int32))])
            acc = [jnp.zeros((L,), jnp.float32) for _ in range(4)]
            acc2 = [jnp.zeros((L,), jnp.float32) for _ in range(4)]
            for d in range(D):
                dfull = jnp.full((L,), d, jnp.int32)
                cvec = plsc.load_gather(rows_v, [rvec, dfull])
                v = plsc.load_gather(code_v, [cvec]) * am
                plsc.store_scatter(out_v, [rvec, dfull], v)
                a = d & 3
                acc[a] = acc[a] + v
                acc2[a] = acc2[a] + v * v
            tot = (acc[0] + acc[1]) + (acc[2] + acc[3])
            tot2 = (acc2[0] + acc2[1]) + (acc2[2] + acc2[3])
            mean = tot * inv_d
            var = tot2 * inv_d - mean * mean
            rstd = _rsqrt_newton(var + eps)
            mstat[pl.ds(g * L, L)] = mean
            rstat[pl.ds(g * L, L)] = rstd

        @pl.loop(0, CHUNK)
        def rowfix(r):
            rfull = jnp.full((L,), r, jnp.int32)
            mv = plsc.load_gather(mstat, [rfull])
            rv = plsc.load_gather(rstat, [rfull])
            for sp in range(D // L):
                dv = out_v[r, pl.ds(sp * L, L)]
                out_v[r, pl.ds(sp * L, L)] = (
                    (dv - mv) * rv * lnw_regs[sp] + lnb_regs[sp])

        pltpu.sync_copy(out_v, out_hbm.at[pl.ds(wid * PER_TILE + j * CHUNK, CHUNK)])


@jax.jit
def _run(x3, weight, am_pad, code, ln_weight, ln_bias):
    mesh = plsc.VectorSubcoreMesh(core_axis_name="c", subcore_axis_name="s")
    return pl.kernel(
        _body,
        out_type=jax.ShapeDtypeStruct((B_TOTAL, D), jnp.float32),
        mesh=mesh,
        compiler_params=pltpu.CompilerParams(
            needs_layout_passes=False, use_tc_tiling_on_sc=False),
        scratch_types=[
            pltpu.VMEM((CHUNK,), jnp.int32),
            pltpu.VMEM((AM_PAD,), jnp.float32),
            pltpu.VMEM((256,), jnp.float32),
            pltpu.VMEM((D,), jnp.float32),
            pltpu.VMEM((D,), jnp.float32),
            pltpu.VMEM((CHUNK, D), jnp.int32),
            pltpu.VMEM((CHUNK, D), jnp.float32),
            pltpu.VMEM((CHUNK,), jnp.float32),
            pltpu.VMEM((CHUNK,), jnp.float32),
            pltpu.SemaphoreType.DMA,
        ],
    )(x3, weight, am_pad, code, ln_weight, ln_bias)


def kernel(x, weight, absmax, code, ln_weight, ln_bias):
    xs = x.shape
    x3 = x.reshape(NW, NCHUNK, CHUNK)
    am_pad = jnp.concatenate(
        [absmax, jnp.zeros((AM_PAD - N_BLOCKS,), jnp.float32)])
    out = _run(x3, weight, am_pad, code, ln_weight, ln_bias)
    return out.reshape(xs[0], xs[1], D)


# speedup vs baseline: 322.7710x; 1.0879x over previous
import jax
import jax.numpy as jnp
from jax import lax
from jax.experimental import pallas as pl
from jax.experimental.pallas import tpu as pltpu
from jax.experimental.pallas import tpu_sc as plsc

V = 1000000
D = 64
BLOCK = 4096
N_BLOCKS = (V * D + BLOCK - 1) // BLOCK
AM_PAD = 16384

NC = 2
NS = 16
NW = NC * NS
L = 16

B_TOTAL = 16384 * 50
PER_TILE = B_TOTAL // NW
CHUNK = 128
NCHUNK = PER_TILE // CHUNK
GROUPS = CHUNK // L
NBUF = 4


def _rsqrt_newton(x):
    half = x * jnp.full((L,), 0.5, jnp.float32)
    i = plsc.bitcast(x, jnp.int32)
    i = jnp.full((L,), 0x5F3759DF, jnp.int32) - lax.shift_right_logical(
        i, jnp.full((L,), 1, jnp.int32))
    y = plsc.bitcast(i, jnp.float32)
    three_half = jnp.full((L,), 1.5, jnp.float32)
    for _ in range(3):
        y = y * (three_half - half * y * y)
    return y


def _body(x_hbm, w_hbm, am_hbm, code_hbm, lnw_hbm, lnb_hbm, out_hbm, *rest):
    idx_v, am_v, code_v, lnw_v, lnb_v = rest[:5]
    rows = rest[5:5 + NBUF]
    outs = rest[5 + NBUF:5 + 2 * NBUF]
    mstat, rstat = rest[5 + 2 * NBUF:7 + 2 * NBUF]
    gsems = rest[7 + 2 * NBUF:7 + 3 * NBUF]
    osems = rest[7 + 3 * NBUF:7 + 4 * NBUF]

    c = lax.axis_index("c")
    s = lax.axis_index("s")
    wid = s * NC + c
    out_base = wid * PER_TILE

    pltpu.sync_copy(x_hbm.at[wid], idx_v)
    pltpu.sync_copy(am_hbm, am_v)
    pltpu.sync_copy(code_hbm, code_v)
    pltpu.sync_copy(lnw_hbm, lnw_v)
    pltpu.sync_copy(lnb_hbm, lnb_v)

    iota = lax.iota(jnp.int32, L)
    inv_d = jnp.full((L,), 1.0 / D, jnp.float32)
    eps = jnp.full((L,), 1e-5, jnp.float32)
    lnw_regs = tuple(lnw_v[pl.ds(sp * L, L)] for sp in range(D // L))
    lnb_regs = tuple(lnb_v[pl.ds(sp * L, L)] for sp in range(D // L))

    def compute_chunk(j, rows_v, out_v):
        @pl.loop(0, GROUPS)
        def group(g):
            rvec = g * L + iota
            idxv = idx_v[j, pl.ds(g * L, L)]
            am = plsc.load_gather(
                am_v,
                [lax.shift_right_logical(idxv, jnp.full((L,), 6, jnp.int32))])
            acc = [jnp.zeros((L,), jnp.float32) for _ in range(4)]
            acc2 = [jnp.zeros((L,), jnp.float32) for _ in range(4)]
            for d in range(D):
                dfull = jnp.full((L,), d, jnp.int32)
                cvec = plsc.load_gather(rows_v, [rvec, dfull])
                v = plsc.load_gather(code_v, [cvec]) * am
                plsc.store_scatter(out_v, [rvec, dfull], v)
                a = d & 3
                acc[a] = acc[a] + v
                acc2[a] = acc2[a] + v * v
            tot = (acc[0] + acc[1]) + (acc[2] + acc[3])
            tot2 = (acc2[0] + acc2[1]) + (acc2[2] + acc2[3])
            mean = tot * inv_d
            var = tot2 * inv_d - mean * mean
            rstd = _rsqrt_newton(var + eps)
            mstat[pl.ds(g * L, L)] = mean
            rstat[pl.ds(g * L, L)] = rstd

        @pl.loop(0, CHUNK)
        def rowfix(r):
            rfull = jnp.full((L,), r, jnp.int32)
            mv = plsc.load_gather(mstat, [rfull])
            rv = plsc.load_gather(rstat, [rfull])
            for sp in range(D // L):
                dv = out_v[r, pl.ds(sp * L, L)]
                out_v[r, pl.ds(sp * L, L)] = (
                    (dv - mv) * rv * lnw_regs[sp] + lnb_regs[sp])

    for b in range(NBUF):
        pltpu.async_copy(w_hbm.at[idx_v.at[b]], rows[b], gsems[b])

    @pl.loop(0, NCHUNK // NBUF)
    def tloop(t):
        for b in range(NBUF):
            j = t * NBUF + b
            pltpu.make_async_copy(
                w_hbm.at[idx_v.at[j]], rows[b], gsems[b]).wait()

            @pl.when(t > 0)
            def _():
                pltpu.make_async_copy(
                    outs[b], out_hbm.at[pl.ds(out_base, CHUNK)],
                    osems[b]).wait()

            compute_chunk(j, rows[b], outs[b])
            pltpu.async_copy(
                outs[b], out_hbm.at[pl.ds(out_base + j * CHUNK, CHUNK)],
                osems[b])
            jp = jnp.minimum(j + NBUF, NCHUNK - 1)
            pltpu.async_copy(w_hbm.at[idx_v.at[jp]], rows[b], gsems[b])

    for b in range(NBUF):
        pltpu.make_async_copy(
            w_hbm.at[idx_v.at[NCHUNK - 1]], rows[b], gsems[b]).wait()
        pltpu.make_async_copy(
            outs[b], out_hbm.at[pl.ds(out_base, CHUNK)], osems[b]).wait()


@jax.jit
def _run(x3, weight, am_pad, code, ln_weight, ln_bias):
    mesh = plsc.VectorSubcoreMesh(core_axis_name="c", subcore_axis_name="s")
    scratch = [
        pltpu.VMEM((NCHUNK, CHUNK), jnp.int32),
        pltpu.VMEM((AM_PAD,), jnp.float32),
        pltpu.VMEM((256,), jnp.float32),
        pltpu.VMEM((D,), jnp.float32),
        pltpu.VMEM((D,), jnp.float32),
    ]
    scratch += [pltpu.VMEM((CHUNK, D), jnp.int32) for _ in range(NBUF)]
    scratch += [pltpu.VMEM((CHUNK, D), jnp.float32) for _ in range(NBUF)]
    scratch += [
        pltpu.VMEM((CHUNK,), jnp.float32),
        pltpu.VMEM((CHUNK,), jnp.float32),
    ]
    scratch += [pltpu.SemaphoreType.DMA for _ in range(2 * NBUF)]
    return pl.kernel(
        _body,
        out_type=jax.ShapeDtypeStruct((B_TOTAL, D), jnp.float32),
        mesh=mesh,
        compiler_params=pltpu.CompilerParams(
            needs_layout_passes=False, use_tc_tiling_on_sc=False),
        scratch_types=scratch,
    )(x3, weight, am_pad, code, ln_weight, ln_bias)


def kernel(x, weight, absmax, code, ln_weight, ln_bias):
    xs = x.shape
    x3 = x.reshape(NW, NCHUNK, CHUNK)
    am_pad = jnp.concatenate(
        [absmax, jnp.zeros((AM_PAD - N_BLOCKS,), jnp.float32)])
    out = _run(x3, weight, am_pad, code, ln_weight, ln_bias)
    return out.reshape(xs[0], xs[1], D)


# speedup vs baseline: 594.7706x; 1.8427x over previous
import jax
import jax.numpy as jnp
import numpy as np
from jax import lax
from jax.experimental import pallas as pl
from jax.experimental.pallas import tpu as pltpu
from jax.experimental.pallas import tpu_sc as plsc

V = 1000000
D = 64
BLOCK = 4096
N_BLOCKS = (V * D + BLOCK - 1) // BLOCK
AM_PAD = 16384

NC = 2
NS = 16
NW = NC * NS
L = 16
NSPAN = D // L

B_TOTAL = 16384 * 50
PER_TILE = B_TOTAL // NW
CHUNK = 128
NCHUNK = PER_TILE // CHUNK
NBUF = 4
RB = 8
NBATCH = CHUNK // RB


def _vf(x):
    return jnp.full((L,), x, jnp.float32)


def _vi(x):
    return jnp.full((L,), x, jnp.int32)


def _xsum(v, perms):
    for perm in perms:
        v = v + v.at[perm].get(mode="promise_in_bounds")
    return v


def _rsqrt_newton(x):
    half = x * _vf(0.5)
    i = plsc.bitcast(x, jnp.int32)
    i = _vi(0x5F3759DF) - lax.shift_right_logical(i, _vi(1))
    y = plsc.bitcast(i, jnp.float32)
    for _ in range(2):
        y = y * (_vf(1.5) - half * y * y)
    return y


def _body(x_hbm, w_hbm, am_hbm, code_hbm, lnw_hbm, lnb_hbm, out_hbm, *rest):
    idx_v, am_v, code_v, lnw_v, lnb_v, crep_v, amc_v = rest[:7]
    rows = rest[7:7 + NBUF]
    outs = rest[7 + NBUF:7 + 2 * NBUF]
    gsems = rest[7 + 2 * NBUF:7 + 3 * NBUF]
    osems = rest[7 + 3 * NBUF:7 + 4 * NBUF]

    c = lax.axis_index("c")
    s = lax.axis_index("s")
    wid = s * NC + c
    out_base = wid * PER_TILE

    pltpu.sync_copy(x_hbm.at[wid], idx_v)
    pltpu.sync_copy(am_hbm, am_v)
    pltpu.sync_copy(code_hbm, code_v)
    pltpu.sync_copy(lnw_hbm, lnw_v)
    pltpu.sync_copy(lnb_hbm, lnb_v)

    iota = lax.iota(jnp.int32, L)

    @pl.loop(0, 256, unroll=4)
    def crep(ci):
        bc = plsc.load_gather(code_v, [jnp.full((L,), ci, jnp.int32)])
        crep_v[pl.ds(ci * L, L)] = bc

    lnw_regs = tuple(lnw_v[pl.ds(sp * L, L)] for sp in range(NSPAN))
    lnb_regs = tuple(lnb_v[pl.ds(sp * L, L)] for sp in range(NSPAN))
    inv_d = _vf(1.0 / D)
    eps = _vf(1e-5)
    perms = tuple(lax.bitwise_xor(iota, _vi(k)) for k in (1, 2, 4, 8))

    def compute_row(r, rows_v, out_v, cspans, amv):
        ci = [lax.shift_left(cs, _vi(4)) + iota for cs in cspans]
        u = [plsc.load_gather(crep_v, [cii]) for cii in ci]
        tot = (u[0] + u[1]) + (u[2] + u[3])
        tot2 = (u[0] * u[0] + u[1] * u[1]) + (u[2] * u[2] + u[3] * u[3])
        sums = _xsum(tot, perms)
        sums2 = _xsum(tot2, perms)
        mean_u = sums * inv_d
        var_u = sums2 * inv_d - mean_u * mean_u
        am2 = amv * amv
        rstd = _rsqrt_newton(var_u * am2 + eps)
        scale = amv * rstd
        for sp in range(NSPAN):
            t = (u[sp] - mean_u) * scale
            out_v[r, pl.ds(sp * L, L)] = t * lnw_regs[sp] + lnb_regs[sp]

    def compute_chunk(j, rows_v, out_v):
        @pl.loop(0, NBATCH)
        def batch(b2):
            rb = b2 * RB
            prev = None
            for r in range(RB + 1):
                if r < RB:
                    rr = rb + r
                    cspans = [rows_v[rr, pl.ds(sp * L, L)]
                              for sp in range(NSPAN)]
                    amv = plsc.load_gather(amc_v, [jnp.full((L,), rr,
                                                            jnp.int32)])
                    cur = (rr, cspans, amv)
                else:
                    cur = None
                if prev is not None:
                    pr, pcs, pam = prev
                    compute_row(pr, rows_v, out_v, pcs, pam)
                prev = cur

    for b in range(NBUF):
        pltpu.async_copy(w_hbm.at[idx_v.at[b]], rows[b], gsems[b])

    @pl.loop(0, NCHUNK // NBUF)
    def tloop(t):
        for b in range(NBUF):
            j = t * NBUF + b

            @pl.loop(0, CHUNK // L)
            def amprep(g):
                idxv = idx_v[j, pl.ds(g * L, L)]
                amv = plsc.load_gather(
                    am_v, [lax.shift_right_logical(idxv, _vi(6))])
                amc_v[pl.ds(g * L, L)] = amv

            pltpu.make_async_copy(
                w_hbm.at[idx_v.at[j]], rows[b], gsems[b]).wait()

            @pl.when(t > 0)
            def _():
                pltpu.make_async_copy(
                    outs[b], out_hbm.at[pl.ds(out_base, CHUNK)],
                    osems[b]).wait()

            compute_chunk(j, rows[b], outs[b])
            pltpu.async_copy(
                outs[b], out_hbm.at[pl.ds(out_base + j * CHUNK, CHUNK)],
                osems[b])
            jp = jnp.minimum(j + NBUF, NCHUNK - 1)
            pltpu.async_copy(w_hbm.at[idx_v.at[jp]], rows[b], gsems[b])

    for b in range(NBUF):
        pltpu.make_async_copy(
            w_hbm.at[idx_v.at[NCHUNK - 1]], rows[b], gsems[b]).wait()
        pltpu.make_async_copy(
            outs[b], out_hbm.at[pl.ds(out_base, CHUNK)], osems[b]).wait()


@jax.jit
def _run(x3, weight, am_pad, code, ln_weight, ln_bias):
    mesh = plsc.VectorSubcoreMesh(core_axis_name="c", subcore_axis_name="s")
    scratch = [
        pltpu.VMEM((NCHUNK, CHUNK), jnp.int32),
        pltpu.VMEM((AM_PAD,), jnp.float32),
        pltpu.VMEM((256,), jnp.float32),
        pltpu.VMEM((D,), jnp.float32),
        pltpu.VMEM((D,), jnp.float32),
        pltpu.VMEM((256 * L,), jnp.float32),
        pltpu.VMEM((CHUNK,), jnp.float32),
    ]
    scratch += [pltpu.VMEM((CHUNK, D), jnp.int32) for _ in range(NBUF)]
    scratch += [pltpu.VMEM((CHUNK, D), jnp.float32) for _ in range(NBUF)]
    scratch += [pltpu.SemaphoreType.DMA for _ in range(2 * NBUF)]
    return pl.kernel(
        _body,
        out_type=jax.ShapeDtypeStruct((B_TOTAL, D), jnp.float32),
        mesh=mesh,
        compiler_params=pltpu.CompilerParams(
            needs_layout_passes=False, use_tc_tiling_on_sc=False),
        scratch_types=scratch,
    )(x3, weight, am_pad, code, ln_weight, ln_bias)


def kernel(x, weight, absmax, code, ln_weight, ln_bias):
    xs = x.shape
    x3 = x.reshape(NW, NCHUNK, CHUNK)
    am_pad = jnp.concatenate(
        [absmax, jnp.zeros((AM_PAD - N_BLOCKS,), jnp.float32)])
    out = _run(x3, weight, am_pad, code, ln_weight, ln_bias)
    return out.reshape(xs[0], xs[1], D)


# speedup vs baseline: 820.4208x; 1.3794x over previous
import jax
import jax.numpy as jnp
import numpy as np
from jax import lax
from jax.experimental import pallas as pl
from jax.experimental.pallas import tpu as pltpu
from jax.experimental.pallas import tpu_sc as plsc

V = 1000000
D = 64
BLOCK = 4096
N_BLOCKS = (V * D + BLOCK - 1) // BLOCK
AM_PAD = 16384

NC = 2
NS = 16
NW = NC * NS
L = 16
NSPAN = D // L

B_TOTAL = 16384 * 50
PER_TILE = B_TOTAL // NW
CHUNK = 128
NCHUNK = PER_TILE // CHUNK
NBUF = 4
RB = 4
NBATCH = CHUNK // RB


def _vf(x):
    return jnp.full((L,), x, jnp.float32)


def _vi(x):
    return jnp.full((L,), x, jnp.int32)


def _xsum(v, perms):
    for perm in perms:
        v = v + v.at[perm].get(mode="promise_in_bounds")
    return v


NEWTON_ITERS = 2


def _rsqrt_newton(x):
    half = x * _vf(0.5)
    i = plsc.bitcast(x, jnp.int32)
    i = _vi(0x5F3759DF) - lax.shift_right_logical(i, _vi(1))
    y = plsc.bitcast(i, jnp.float32)
    for _ in range(NEWTON_ITERS):
        y = y * (_vf(1.5) - half * y * y)
    return y


def _body(x_hbm, w_hbm, am_hbm, code_hbm, lnw_hbm, lnb_hbm, out_hbm, *rest):
    idx_v, am_v, code_v, lnw_v, lnb_v, crep_v, amc_v = rest[:7]
    rows = rest[7:7 + NBUF]
    outs = rest[7 + NBUF:7 + 2 * NBUF]
    gsems = rest[7 + 2 * NBUF:7 + 3 * NBUF]
    osems = rest[7 + 3 * NBUF:7 + 4 * NBUF]

    c = lax.axis_index("c")
    s = lax.axis_index("s")
    wid = s * NC + c
    out_base = wid * PER_TILE

    pltpu.sync_copy(x_hbm.at[wid], idx_v)
    pltpu.sync_copy(am_hbm, am_v)
    pltpu.sync_copy(code_hbm, code_v)
    pltpu.sync_copy(lnw_hbm, lnw_v)
    pltpu.sync_copy(lnb_hbm, lnb_v)

    iota = lax.iota(jnp.int32, L)

    @pl.loop(0, 256, unroll=4)
    def crep(ci):
        bc = plsc.load_gather(code_v, [jnp.full((L,), ci, jnp.int32)])
        crep_v[pl.ds(ci * L, L)] = bc

    lnw_regs = tuple(lnw_v[pl.ds(sp * L, L)] for sp in range(NSPAN))
    lnb_regs = tuple(lnb_v[pl.ds(sp * L, L)] for sp in range(NSPAN))
    inv_d = _vf(1.0 / D)
    eps = _vf(1e-5)
    perms = tuple(lax.bitwise_xor(iota, _vi(k)) for k in (1, 2, 4, 8))

    def compute_chunk(j, rows_v, out_v):
        @pl.loop(0, NBATCH)
        def batch(b2):
            rb = b2 * RB
            rrs = [rb + r for r in range(RB)]
            cs = [[rows_v[rr, pl.ds(sp * L, L)] for sp in range(NSPAN)]
                  for rr in rrs]
            amv = [plsc.load_gather(amc_v, [jnp.full((L,), rr, jnp.int32)])
                   for rr in rrs]
            ci = [[lax.shift_left(cs[r][sp], _vi(4)) + iota
                   for sp in range(NSPAN)] for r in range(RB)]
            u = [[plsc.load_gather(crep_v, [ci[r][sp]])
                  for sp in range(NSPAN)] for r in range(RB)]
            t01 = [u[r][0] + u[r][1] for r in range(RB)]
            t23 = [u[r][2] + u[r][3] for r in range(RB)]
            sq = [[u[r][sp] * u[r][sp] for sp in range(NSPAN)]
                  for r in range(RB)]
            q01 = [sq[r][0] + sq[r][1] for r in range(RB)]
            q23 = [sq[r][2] + sq[r][3] for r in range(RB)]
            vv = [t01[r] + t23[r] for r in range(RB)] + \
                 [q01[r] + q23[r] for r in range(RB)]
            for perm in perms:
                pv = [v.at[perm].get(mode="promise_in_bounds") for v in vv]
                vv = [v + p for v, p in zip(vv, pv)]
            mean = [s * inv_d for s in vv[:RB]]
            e2 = [s * inv_d for s in vv[RB:]]
            mm = [m * m for m in mean]
            var = [e - m2 for e, m2 in zip(e2, mm)]
            am2 = [a * a for a in amv]
            ve = [v * a2 + eps for v, a2 in zip(var, am2)]
            half = [v * _vf(0.5) for v in ve]
            iv = [plsc.bitcast(v, jnp.int32) for v in ve]
            iv = [_vi(0x5F3759DF) - lax.shift_right_logical(i2, _vi(1))
                  for i2 in iv]
            y = [plsc.bitcast(i2, jnp.float32) for i2 in iv]
            for _ in range(NEWTON_ITERS):
                yy = [yi * yi for yi in y]
                hyy = [h * t for h, t in zip(half, yy)]
                th = [_vf(1.5) - t for t in hyy]
                y = [yi * t for yi, t in zip(y, th)]
            scale = [a * yi for a, yi in zip(amv, y)]
            for sp in range(NSPAN):
                t = [u[r][sp] - mean[r] for r in range(RB)]
                t = [tt * scale[r] for r, tt in enumerate(t)]
                t = [tt * lnw_regs[sp] for tt in t]
                t = [tt + lnb_regs[sp] for tt in t]
                for r in range(RB):
                    out_v[rrs[r], pl.ds(sp * L, L)] = t[r]

    for b in range(NBUF):
        pltpu.async_copy(w_hbm.at[idx_v.at[b]], rows[b], gsems[b])

    @pl.loop(0, NCHUNK // NBUF)
    def tloop(t):
        for b in range(NBUF):
            j = t * NBUF + b

            @pl.loop(0, CHUNK // L)
            def amprep(g):
                idxv = idx_v[j, pl.ds(g * L, L)]
                amv = plsc.load_gather(
                    am_v, [lax.shift_right_logical(idxv, _vi(6))])
                amc_v[pl.ds(g * L, L)] = amv

            pltpu.make_async_copy(
                w_hbm.at[idx_v.at[j]], rows[b], gsems[b]).wait()

            @pl.when(t > 0)
            def _():
                pltpu.make_async_copy(
                    outs[b], out_hbm.at[pl.ds(out_base, CHUNK)],
                    osems[b]).wait()

            compute_chunk(j, rows[b], outs[b])
            pltpu.async_copy(
                outs[b], out_hbm.at[pl.ds(out_base + j * CHUNK, CHUNK)],
                osems[b])
            jp = jnp.minimum(j + NBUF, NCHUNK - 1)
            pltpu.async_copy(w_hbm.at[idx_v.at[jp]], rows[b], gsems[b])

    for b in range(NBUF):
        pltpu.make_async_copy(
            w_hbm.at[idx_v.at[NCHUNK - 1]], rows[b], gsems[b]).wait()
        pltpu.make_async_copy(
            outs[b], out_hbm.at[pl.ds(out_base, CHUNK)], osems[b]).wait()


@jax.jit
def _run(x3, weight, am_pad, code, ln_weight, ln_bias):
    mesh = plsc.VectorSubcoreMesh(core_axis_name="c", subcore_axis_name="s")
    scratch = [
        pltpu.VMEM((NCHUNK, CHUNK), jnp.int32),
        pltpu.VMEM((AM_PAD,), jnp.float32),
        pltpu.VMEM((256,), jnp.float32),
        pltpu.VMEM((D,), jnp.float32),
        pltpu.VMEM((D,), jnp.float32),
        pltpu.VMEM((256 * L,), jnp.float32),
        pltpu.VMEM((CHUNK,), jnp.float32),
    ]
    scratch += [pltpu.VMEM((CHUNK, D), jnp.int32) for _ in range(NBUF)]
    scratch += [pltpu.VMEM((CHUNK, D), jnp.float32) for _ in range(NBUF)]
    scratch += [pltpu.SemaphoreType.DMA for _ in range(2 * NBUF)]
    return pl.kernel(
        _body,
        out_type=jax.ShapeDtypeStruct((B_TOTAL, D), jnp.float32),
        mesh=mesh,
        compiler_params=pltpu.CompilerParams(
            needs_layout_passes=False, use_tc_tiling_on_sc=False),
        scratch_types=scratch,
    )(x3, weight, am_pad, code, ln_weight, ln_bias)


def kernel(x, weight, absmax, code, ln_weight, ln_bias):
    xs = x.shape
    x3 = x.reshape(NW, NCHUNK, CHUNK)
    am_pad = jnp.concatenate(
        [absmax, jnp.zeros((AM_PAD - N_BLOCKS,), jnp.float32)])
    out = _run(x3, weight, am_pad, code, ln_weight, ln_bias)
    return out.reshape(xs[0], xs[1], D)
